# baseline (speedup 1.0000x reference)
import dataclasses
import functools

import jax
import jax.numpy as jnp
import numpy as np
from jax import lax
from jax.experimental import pallas as pl
from jax.experimental.pallas import tpu as pltpu
from jax.experimental.pallas import tpu_sc as plsc

L = 16
BLOCK = 3200
REP = 16

INV_H = np.float32(6.3)
SCALE = np.float32(1.03e-10)

_CP = pltpu.CompilerParams()
_flds = pltpu.CompilerParams.__dataclass_fields__
if "needs_layout_passes" in _flds:
    _CP = dataclasses.replace(_CP, needs_layout_passes=False)
if "use_tc_tiling_on_sc" in _flds:
    _CP = dataclasses.replace(_CP, use_tc_tiling_on_sc=False)


def _interp(t, tab_ref, lane):
    ti = jnp.minimum(t.astype(jnp.int32), 62)
    w = t - ti.astype(jnp.float32)
    y0 = plsc.load_gather(tab_ref, [ti, lane])
    y1 = plsc.load_gather(tab_ref, [ti + 1, lane])
    return y0 * (1.0 - w) + y1 * w


def kernel(Av, y_in, x_CO, theta_CO, x_H2, theta_H2, x_Av, theta_Av):
    N = Av.shape[0]
    rep = lambda t: jnp.broadcast_to(t[:, None], (t.shape[0], REP))
    mesh = plsc.VectorSubcoreMesh(core_axis_name="c", subcore_axis_name="s")

    @functools.partial(
        pl.kernel,
        out_type=jax.ShapeDtypeStruct((N,), jnp.float32),
        mesh=mesh,
        compiler_params=_CP,
        scratch_types=[
            pltpu.VMEM((64, REP), jnp.float32),
            pltpu.VMEM((64, REP), jnp.float32),
            pltpu.VMEM((64, REP), jnp.float32),
        ],
    )
    def sc_kernel(av_hbm, y_hbm, tco_hbm, th2_hbm, tav_hbm, out_hbm,
                  tco_v, th2_v, tav_v):
        pltpu.sync_copy(tco_hbm, tco_v)
        pltpu.sync_copy(th2_hbm, th2_v)
        pltpu.sync_copy(tav_hbm, tav_v)

        def body(av_b, y_b, out_b):
            @pl.loop(0, BLOCK, step=L)
            def _(i):
                lane = lax.iota(jnp.int32, L)
                rows = lane + i
                zero = jnp.zeros((L,), jnp.int32)
                a = plsc.load_gather(av_b, [rows, zero])
                yco = plsc.load_gather(
                    y_b, [rows, jnp.full((L,), 5, jnp.int32)])
                yh2 = plsc.load_gather(
                    y_b, [rows, jnp.full((L,), 2, jnp.int32)])
                s_co = _interp(a * yco * INV_H, tco_v, lane)
                s_h2 = _interp(a * yh2 * INV_H, th2_v, lane)
                s_av = _interp(a * INV_H, tav_v, lane)
                out_b[pl.ds(i, L)] = SCALE * s_co * s_h2 * s_av

        pltpu.emit_pipeline(
            body,
            grid=(N // BLOCK,),
            in_specs=[
                pl.BlockSpec((BLOCK, 1), lambda i: (i, 0)),
                pl.BlockSpec((BLOCK, 8), lambda i: (i, 0)),
            ],
            out_specs=[pl.BlockSpec((BLOCK,), lambda i: (i,))],
            core_axis_name=("c", "s"),
            dimension_semantics=(pltpu.PARALLEL,),
        )(av_hbm, y_hbm, out_hbm)

    out = sc_kernel(Av, y_in, rep(theta_CO), rep(theta_H2), rep(theta_Av))
    return out.reshape(N, 1)


# speedup vs baseline: 2.4992x; 2.4992x over previous
import dataclasses
import functools

import jax
import jax.numpy as jnp
import numpy as np
from jax import lax
from jax.experimental import pallas as pl
from jax.experimental.pallas import tpu as pltpu
from jax.experimental.pallas import tpu_sc as plsc

L = 16
BLOCK = 128
REP = 16

INV_H = np.float32(6.3)
SCALE = np.float32(1.03e-10)

_CP = pltpu.CompilerParams()
_flds = pltpu.CompilerParams.__dataclass_fields__
if "needs_layout_passes" in _flds:
    _CP = dataclasses.replace(_CP, needs_layout_passes=False)


def _interp(t, val_ref, slope_ref, lane):
    ti = t.astype(jnp.int32)
    w = t - ti.astype(jnp.float32)
    y0 = plsc.load_gather(val_ref, [ti, lane])
    dy = plsc.load_gather(slope_ref, [ti, lane])
    return y0 + w * dy


def kernel(Av, y_in, x_CO, theta_CO, x_H2, theta_H2, x_Av, theta_Av):
    N = Av.shape[0]

    def tables(theta, scale=np.float32(1.0)):
        v = theta * scale
        s = jnp.concatenate([v[1:] - v[:-1], jnp.zeros((1,), jnp.float32)])
        rep = lambda t: jnp.broadcast_to(t[:, None], (t.shape[0], REP))
        return rep(v), rep(s)

    vco, sco = tables(theta_CO)
    vh2, sh2 = tables(theta_H2)
    vav, sav = tables(theta_Av, SCALE)

    mesh = plsc.VectorSubcoreMesh(core_axis_name="c", subcore_axis_name="s")

    @functools.partial(
        pl.kernel,
        out_type=jax.ShapeDtypeStruct((N,), jnp.float32),
        mesh=mesh,
        compiler_params=_CP,
        scratch_types=[
            pltpu.VMEM((64, REP), jnp.float32),
            pltpu.VMEM((64, REP), jnp.float32),
            pltpu.VMEM((64, REP), jnp.float32),
            pltpu.VMEM((64, REP), jnp.float32),
            pltpu.VMEM((64, REP), jnp.float32),
            pltpu.VMEM((64, REP), jnp.float32),
        ],
    )
    def sc_kernel(av_hbm, y_hbm, vco_h, sco_h, vh2_h, sh2_h, vav_h, sav_h,
                  out_hbm, vco_v, sco_v, vh2_v, sh2_v, vav_v, sav_v):
        pltpu.sync_copy(vco_h, vco_v)
        pltpu.sync_copy(sco_h, sco_v)
        pltpu.sync_copy(vh2_h, vh2_v)
        pltpu.sync_copy(sh2_h, sh2_v)
        pltpu.sync_copy(vav_h, vav_v)
        pltpu.sync_copy(sav_h, sav_v)

        def body(av_b, y_b, out_b):
            @pl.loop(0, BLOCK, step=L)
            def _(i):
                lane = lax.iota(jnp.int32, L)
                rows = lane + i
                zero = jnp.zeros((L,), jnp.int32)
                a = plsc.load_gather(av_b, [rows, zero])
                yco = plsc.load_gather(
                    y_b, [rows, jnp.full((L,), 5, jnp.int32)])
                yh2 = plsc.load_gather(
                    y_b, [rows, jnp.full((L,), 2, jnp.int32)])
                s_co = _interp(a * yco * INV_H, vco_v, sco_v, lane)
                s_h2 = _interp(a * yh2 * INV_H, vh2_v, sh2_v, lane)
                s_av = _interp(a * INV_H, vav_v, sav_v, lane)
                out_b[pl.ds(i, L)] = s_co * s_h2 * s_av

        pltpu.emit_pipeline(
            body,
            grid=(N // BLOCK,),
            in_specs=[
                pl.BlockSpec((BLOCK, 1), lambda i: (i, 0)),
                pl.BlockSpec((BLOCK, 32), lambda i: (i, 0)),
            ],
            out_specs=[pl.BlockSpec((BLOCK,), lambda i: (i,))],
            core_axis_name=("c", "s"),
            dimension_semantics=(pltpu.PARALLEL,),
        )(av_hbm, y_hbm, out_hbm)

    out = sc_kernel(Av, y_in, vco, sco, vh2, sh2, vav, sav)
    return out.reshape(N, 1)


# speedup vs baseline: 12.7419x; 5.0985x over previous
import dataclasses
import functools

import jax
import jax.numpy as jnp
import numpy as np
from jax import lax
from jax.experimental import pallas as pl
from jax.experimental.pallas import tpu as pltpu
from jax.experimental.pallas import tpu_sc as plsc

L = 16
BLOCK = 8000
REP = 16

INV_H = np.float32(6.3)
SCALE = np.float32(1.03e-10)

_CP = pltpu.CompilerParams()
_flds = pltpu.CompilerParams.__dataclass_fields__
if "needs_layout_passes" in _flds:
    _CP = dataclasses.replace(_CP, needs_layout_passes=False)


def _interp(t, val_ref, slope_ref, lane):
    ti = t.astype(jnp.int32)
    w = t - ti.astype(jnp.float32)
    y0 = plsc.load_gather(val_ref, [ti, lane])
    dy = plsc.load_gather(slope_ref, [ti, lane])
    return y0 + w * dy


def kernel(Av, y_in, x_CO, theta_CO, x_H2, theta_H2, x_Av, theta_Av):
    N = Av.shape[0]
    av_flat = lax.squeeze(Av, (1,))
    y5 = lax.squeeze(lax.slice(y_in, (0, 5), (N, 6)), (1,))
    y2 = lax.squeeze(lax.slice(y_in, (0, 2), (N, 3)), (1,))

    def tables(theta, scale=np.float32(1.0)):
        v = theta * scale
        s = jnp.concatenate([v[1:] - v[:-1], jnp.zeros((1,), jnp.float32)])
        rep = lambda t: jnp.broadcast_to(t[:, None], (t.shape[0], REP))
        return rep(v), rep(s)

    vco, sco = tables(theta_CO)
    vh2, sh2 = tables(theta_H2)
    vav, sav = tables(theta_Av, SCALE)

    mesh = plsc.VectorSubcoreMesh(core_axis_name="c", subcore_axis_name="s")

    @functools.partial(
        pl.kernel,
        out_type=jax.ShapeDtypeStruct((N,), jnp.float32),
        mesh=mesh,
        compiler_params=_CP,
        scratch_types=[
            pltpu.VMEM((64, REP), jnp.float32),
            pltpu.VMEM((64, REP), jnp.float32),
            pltpu.VMEM((64, REP), jnp.float32),
            pltpu.VMEM((64, REP), jnp.float32),
            pltpu.VMEM((64, REP), jnp.float32),
            pltpu.VMEM((64, REP), jnp.float32),
        ],
    )
    def sc_kernel(av_hbm, y5_hbm, y2_hbm, vco_h, sco_h, vh2_h, sh2_h,
                  vav_h, sav_h, out_hbm,
                  vco_v, sco_v, vh2_v, sh2_v, vav_v, sav_v):
        pltpu.sync_copy(vco_h, vco_v)
        pltpu.sync_copy(sco_h, sco_v)
        pltpu.sync_copy(vh2_h, vh2_v)
        pltpu.sync_copy(sh2_h, sh2_v)
        pltpu.sync_copy(vav_h, vav_v)
        pltpu.sync_copy(sav_h, sav_v)

        def body(av_b, y5_b, y2_b, out_b):
            @pl.loop(0, BLOCK, step=L)
            def _(i):
                lane = lax.iota(jnp.int32, L)
                a = av_b[pl.ds(i, L)]
                yco = y5_b[pl.ds(i, L)]
                yh2 = y2_b[pl.ds(i, L)]
                s_co = _interp(a * yco * INV_H, vco_v, sco_v, lane)
                s_h2 = _interp(a * yh2 * INV_H, vh2_v, sh2_v, lane)
                s_av = _interp(a * INV_H, vav_v, sav_v, lane)
                out_b[pl.ds(i, L)] = s_co * s_h2 * s_av

        pltpu.emit_pipeline(
            body,
            grid=(N // BLOCK,),
            in_specs=[
                pl.BlockSpec((BLOCK,), lambda i: (i,)),
                pl.BlockSpec((BLOCK,), lambda i: (i,)),
                pl.BlockSpec((BLOCK,), lambda i: (i,)),
            ],
            out_specs=[pl.BlockSpec((BLOCK,), lambda i: (i,))],
            core_axis_name=("c", "s"),
            dimension_semantics=(pltpu.PARALLEL,),
        )(av_hbm, y5_hbm, y2_hbm, out_hbm)

    out = sc_kernel(av_flat, y5, y2, vco, sco, vh2, sh2, vav, sav)
    return out.reshape(N, 1)


# speedup vs baseline: 17.6857x; 1.3880x over previous
import dataclasses
import functools

import jax
import jax.numpy as jnp
import numpy as np
from jax import lax
from jax.experimental import pallas as pl
from jax.experimental.pallas import tpu as pltpu
from jax.experimental.pallas import tpu_sc as plsc

L = 16
BLOCK = 3200
REP = 16

INV_H = np.float32(6.3)
SCALE = np.float32(1.03e-10)

_CP = pltpu.CompilerParams()
_flds = pltpu.CompilerParams.__dataclass_fields__
if "needs_layout_passes" in _flds:
    _CP = dataclasses.replace(_CP, needs_layout_passes=False)
if "use_tc_tiling_on_sc" in _flds:
    _CP = dataclasses.replace(_CP, use_tc_tiling_on_sc=True)


def _interp(t, val_ref, slope_ref, lane):
    ti = t.astype(jnp.int32)
    w = t - ti.astype(jnp.float32)
    y0 = plsc.load_gather(val_ref, [ti, lane])
    dy = plsc.load_gather(slope_ref, [ti, lane])
    return y0 + w * dy


def kernel(Av, y_in, x_CO, theta_CO, x_H2, theta_H2, x_Av, theta_Av):
    N = Av.shape[0]
    av_flat = lax.reshape(lax.transpose(Av, (1, 0)), (N,))
    y_t = lax.transpose(y_in, (1, 0))

    def tables(theta, scale=np.float32(1.0)):
        v = theta * scale
        s = jnp.concatenate([v[1:] - v[:-1], jnp.zeros((1,), jnp.float32)])
        rep = lambda t: jnp.broadcast_to(t[:, None], (t.shape[0], REP))
        return rep(v), rep(s)

    vco, sco = tables(theta_CO)
    vh2, sh2 = tables(theta_H2)
    vav, sav = tables(theta_Av, SCALE)

    mesh = plsc.VectorSubcoreMesh(core_axis_name="c", subcore_axis_name="s")

    @functools.partial(
        pl.kernel,
        out_type=jax.ShapeDtypeStruct((N,), jnp.float32),
        mesh=mesh,
        compiler_params=_CP,
        scratch_types=[
            pltpu.VMEM((64, REP), jnp.float32),
            pltpu.VMEM((64, REP), jnp.float32),
            pltpu.VMEM((64, REP), jnp.float32),
            pltpu.VMEM((64, REP), jnp.float32),
            pltpu.VMEM((64, REP), jnp.float32),
            pltpu.VMEM((64, REP), jnp.float32),
        ],
    )
    def sc_kernel(av_hbm, y_hbm, vco_h, sco_h, vh2_h, sh2_h, vav_h, sav_h,
                  out_hbm, vco_v, sco_v, vh2_v, sh2_v, vav_v, sav_v):
        pltpu.sync_copy(vco_h, vco_v)
        pltpu.sync_copy(sco_h, sco_v)
        pltpu.sync_copy(vh2_h, vh2_v)
        pltpu.sync_copy(sh2_h, sh2_v)
        pltpu.sync_copy(vav_h, vav_v)
        pltpu.sync_copy(sav_h, sav_v)

        def body(av_b, y_b, out_b):
            @pl.loop(0, BLOCK, step=L)
            def _(i):
                lane = lax.iota(jnp.int32, L)
                cols = lane + i
                a = av_b[pl.ds(i, L)]
                yco = plsc.load_gather(
                    y_b, [jnp.full((L,), 5, jnp.int32), cols])
                yh2 = plsc.load_gather(
                    y_b, [jnp.full((L,), 2, jnp.int32), cols])
                s_co = _interp(a * yco * INV_H, vco_v, sco_v, lane)
                s_h2 = _interp(a * yh2 * INV_H, vh2_v, sh2_v, lane)
                s_av = _interp(a * INV_H, vav_v, sav_v, lane)
                out_b[pl.ds(i, L)] = s_co * s_h2 * s_av

        pltpu.emit_pipeline(
            body,
            grid=(N // BLOCK,),
            in_specs=[
                pl.BlockSpec((BLOCK,), lambda i: (i,)),
                pl.BlockSpec((8, BLOCK), lambda i: (0, i)),
            ],
            out_specs=[pl.BlockSpec((BLOCK,), lambda i: (i,))],
            core_axis_name=("c", "s"),
            dimension_semantics=(pltpu.PARALLEL,),
        )(av_hbm, y_hbm, out_hbm)

    out = sc_kernel(av_flat, y_t, vco, sco, vh2, sh2, vav, sav)
    return out.reshape(N, 1)
